# Initial kernel scaffold; baseline (speedup 1.0000x reference)
import jax
import jax.numpy as jnp
from jax.experimental import pallas as pl


def kernel(x):
    raise NotImplementedError("write your pallas kernel here")


# speedup vs baseline: 215.0527x; 215.0527x over previous
import functools

import jax
import jax.numpy as jnp
from jax import lax
from jax.experimental import pallas as pl
from jax.experimental.pallas import tpu as pltpu
from jax.experimental.pallas import tpu_sc as plsc

H = 224
W = 224
HO = 112
WO = 112
PH = 225
PW = 232
NW = 32
MAGIC = float(2 ** 23)


def _mode_row(xpad, out_v, r):
    lane2 = lax.iota(jnp.int32, 16) * 2
    rows = [jnp.full((16,), jnp.where(2 * r + i - 1 < 0, H, 2 * r + i - 1),
                     jnp.int32) for i in range(3)]
    lane0 = lax.iota(jnp.int32, 16) == 0
    for v in range(7):
        col0 = lane2 + (32 * v + 7)
        cols = [col0, col0 + 1, col0 + 2]
        bins = []
        for i in range(3):
            for j in range(3):
                ri, cj = rows[i], cols[j]
                if v == 0 and j == 0:
                    ri = jnp.where(lane0, H, ri)
                    cj = jnp.where(lane0, 8, cj)
                val = plsc.load_gather(xpad, [ri, cj])
                bins.append((val * 16.0 + MAGIC) - MAGIC)
        scores = [32.0 - b for b in bins]
        for i in range(9):
            for j in range(i + 1, 9):
                c = jnp.where(bins[i] == bins[j], 32.0, 0.0)
                scores[i] = scores[i] + c
                scores[j] = scores[j] + c
        m = scores[0]
        for i in range(1, 9):
            m = jnp.maximum(m, scores[i])
        mode_bin = (-(m.astype(jnp.int32))) & 31
        out_v[r, pl.ds(16 * v, 16)] = mode_bin.astype(jnp.float32) * 0.0625


def kernel(x):
    B, C, Hx, Wx = x.shape
    n_img = B * C
    ipw = n_img // NW
    xf = x.reshape(n_img, Hx, Wx)
    mesh = plsc.VectorSubcoreMesh(core_axis_name="c", subcore_axis_name="s")

    @functools.partial(
        pl.kernel,
        out_type=jax.ShapeDtypeStruct((n_img, HO, WO), jnp.float32),
        mesh=mesh,
        scratch_types=[
            pltpu.VMEM((PH, PW), jnp.float32),
            pltpu.VMEM((HO, WO), jnp.float32),
        ],
        compiler_params=pltpu.CompilerParams(
            needs_layout_passes=False, use_tc_tiling_on_sc=False),
    )
    def run(x_hbm, out_hbm, xpad, out_v):
        wid = lax.axis_index("s") * 2 + lax.axis_index("c")
        lane = lax.iota(jnp.int32, 16)
        zval = lane.astype(jnp.float32) * 0.0
        for base in range(8, 224, 16):
            xpad[H, pl.ds(base, 16)] = zval

        def per_image(t, carry):
            img = wid * ipw + t
            pltpu.sync_copy(x_hbm.at[img], xpad.at[pl.ds(0, H), pl.ds(8, W)])

            def per_row(r, c2):
                _mode_row(xpad, out_v, r)
                return c2

            lax.fori_loop(0, HO, per_row, 0)
            pltpu.sync_copy(out_v, out_hbm.at[img])
            return carry

        lax.fori_loop(0, ipw, per_image, 0)

    return run(xf).reshape(B, C, HO, WO)


# speedup vs baseline: 318.7305x; 1.4821x over previous
import functools

import jax
import jax.numpy as jnp
import numpy as np
from jax import lax
from jax.experimental import pallas as pl
from jax.experimental.pallas import tpu as pltpu
from jax.experimental.pallas import tpu_sc as plsc

H = 224
W = 224
HO = 112
WO = 112
PH = 225
PW = 232
NW = 32
MAGIC = float(2 ** 23)
EXP23 = 0x4B000000
I16_32 = np.int16(32)
I16_0 = np.int16(0)


def _col_indices():
    lane2 = lax.iota(jnp.int32, 16) * 2
    lane0 = lax.iota(jnp.int32, 16) == 0
    cols = []
    for v in range(7):
        col0 = lane2 + (32 * v + 7)
        if v == 0:
            cols.append([jnp.where(lane0, 8, col0), col0 + 1, col0 + 2])
        else:
            cols.append([col0, col0 + 1, col0 + 2])
    return lane0, cols


def _mode_two_rows(xpad, out_v, u, lane0, cols_all):
    r_top = jnp.where(4 * u - 1 < 0, H, 4 * u - 1)
    rows = [jnp.full((16,), r_top, jnp.int32)] + [
        jnp.full((16,), 4 * u + k, jnp.int32) for k in range(4)
    ]
    for v in range(7):
        cols = cols_all[v]
        bin32 = []
        for k in range(5):
            brow = []
            for j in range(3):
                rk, cj = rows[k], cols[j]
                if v == 0 and j == 0:
                    rk = jnp.where(lane0, H, rk)
                val = plsc.load_gather(xpad, [rk, cj])
                t = val * 16.0 + MAGIC
                brow.append(plsc.bitcast(t, jnp.int32) - EXP23)
            bin32.append(brow)
        p = [
            plsc.pack(bin32[i][j], bin32[i + 2][j],
                      format=plsc.PackFormat.INTERLEAVED,
                      preferred_element_type=jnp.int16)
            for i in range(3) for j in range(3)
        ]
        scores = [I16_32 - b for b in p]
        for i in range(9):
            for j in range(i + 1, 9):
                c = jnp.where(p[i] == p[j], I16_32, I16_0)
                scores[i] = scores[i] + c
                scores[j] = scores[j] + c
        m = scores[0]
        for i in range(1, 9):
            m = jnp.maximum(m, scores[i])
        ma, mb = plsc.unpack(m, format=plsc.PackFormat.INTERLEAVED,
                             preferred_element_type=jnp.int32)
        out_v[2 * u, pl.ds(16 * v, 16)] = \
            ((-ma) & 31).astype(jnp.float32) * 0.0625
        out_v[2 * u + 1, pl.ds(16 * v, 16)] = \
            ((-mb) & 31).astype(jnp.float32) * 0.0625


def kernel(x):
    B, C, Hx, Wx = x.shape
    n_img = B * C
    ipw = n_img // NW
    xf = x.reshape(n_img, Hx, Wx)
    mesh = plsc.VectorSubcoreMesh(core_axis_name="c", subcore_axis_name="s")

    @functools.partial(
        pl.kernel,
        out_type=jax.ShapeDtypeStruct((n_img, HO, WO), jnp.float32),
        mesh=mesh,
        scratch_types=[
            pltpu.VMEM((PH, PW), jnp.float32),
            pltpu.VMEM((PH, PW), jnp.float32),
            pltpu.VMEM((HO, WO), jnp.float32),
            pltpu.SemaphoreType.DMA,
            pltpu.SemaphoreType.DMA,
        ],
        compiler_params=pltpu.CompilerParams(
            needs_layout_passes=False, use_tc_tiling_on_sc=False),
    )
    def run(x_hbm, out_hbm, xpad_a, xpad_b, out_v, sem_a, sem_b):
        wid = lax.axis_index("s") * 2 + lax.axis_index("c")
        base_img = wid * ipw
        last_img = base_img + ipw - 1
        lane = lax.iota(jnp.int32, 16)
        zval = lane.astype(jnp.float32) * 0.0
        for base in range(8, 224, 16):
            xpad_a[H, pl.ds(base, 16)] = zval
            xpad_b[H, pl.ds(base, 16)] = zval

        def start_in(img, buf, sem):
            pltpu.make_async_copy(
                x_hbm.at[img], buf.at[pl.ds(0, H), pl.ds(8, W)], sem).start()

        def wait_in(buf, sem):
            pltpu.make_async_copy(
                x_hbm.at[0], buf.at[pl.ds(0, H), pl.ds(8, W)], sem).wait()

        lane0, cols_all = _col_indices()

        def compute(buf, img):
            def per_row_pair(u, c2):
                _mode_two_rows(buf, out_v, u, lane0, cols_all)
                return c2

            lax.fori_loop(0, HO // 2, per_row_pair, 0)
            pltpu.sync_copy(out_v, out_hbm.at[img])

        start_in(base_img, xpad_a, sem_a)

        def per_pair(s, carry):
            img = base_img + 2 * s
            wait_in(xpad_a, sem_a)
            start_in(jnp.minimum(img + 1, last_img), xpad_b, sem_b)
            compute(xpad_a, img)
            wait_in(xpad_b, sem_b)
            start_in(jnp.minimum(img + 2, last_img), xpad_a, sem_a)
            compute(xpad_b, img + 1)
            return carry

        lax.fori_loop(0, ipw // 2, per_pair, 0)
        wait_in(xpad_a, sem_a)

    return run(xf).reshape(B, C, HO, WO)
